# baseline (speedup 1.0000x reference)
import functools

import jax
import jax.numpy as jnp
from jax import lax
from jax.experimental import pallas as pl
from jax.experimental.pallas import tpu as pltpu
from jax.experimental.pallas import tpu_sc as plsc

NC = 2
NS = 16
NW = NC * NS
L = 16
F = 16
WCHUNK = 64
NRING = 3


def _task_kernel_body(task_hbm, tfT_hbm, trows_hbm,
                      tidx_v, tring_v, trows_v, sems):
    b_per_w = tidx_v.shape[0]
    n_groups = b_per_w // L
    wid = lax.axis_index("s") * NC + lax.axis_index("c")
    base = wid * b_per_w

    pltpu.sync_copy(task_hbm.at[pl.ds(base, b_per_w)], tidx_v)

    row_iota = lax.iota(jnp.int32, L)

    def t_fire(g, ring):
        tv = tidx_v[pl.ds(g * L, L)]
        for j in range(L):
            blk = pl.multiple_of(
                lax.shift_right_logical(tv[j], 7) * 128, 128)
            pltpu.async_copy(tfT_hbm.at[:, pl.ds(blk, 128)],
                             tring_v.at[ring, j], sems.at[ring])

    def t_drain(ring):
        for j in range(L):
            pltpu.make_async_copy(tfT_hbm.at[:, pl.ds(0, 128)],
                                  tring_v.at[ring, j], sems.at[ring]).wait()

    for r in range(NRING):
        t_fire(r, r)

    def t_group(g, ring):
        t_drain(ring)
        tv = tidx_v[pl.ds(g * L, L)]
        for j in range(L):
            col = lax.bitwise_and(tv[j], 127)
            cidx = jnp.full((L,), col, jnp.int32)
            tcol = plsc.load_gather(tring_v.at[ring, j], [row_iota, cidx])
            trows_v[pl.ds((g * L + j) * F, F)] = tcol

        @pl.when(g + NRING < n_groups)
        def _():
            t_fire(g + NRING, ring)

    @pl.loop(0, n_groups, step=NRING)
    def _(g):
        for r in range(NRING):
            @pl.when(g + r < n_groups)
            def _(r=r):
                t_group(g + r, r)

    pltpu.sync_copy(trows_v, trows_hbm.at[wid])


def _dot_kernel_body(worker_hbm, wfp_hbm, trows_hbm, out_hbm,
                     widx_v, wblk_v, wbuf_v, trows_v, wrows_v, out_v, sems):
    b_per_w = widx_v.shape[0]
    n_wchunks = b_per_w // WCHUNK
    wid = lax.axis_index("s") * NC + lax.axis_index("c")
    base = wid * b_per_w

    pltpu.sync_copy(worker_hbm.at[pl.ds(base, b_per_w)], widx_v)
    pltpu.sync_copy(trows_hbm.at[wid], trows_v)

    row_iota = lax.iota(jnp.int32, L)

    @pl.loop(0, b_per_w, step=L)
    def _(g):
        wblk_v[pl.ds(g, L)] = lax.shift_right_logical(widx_v[pl.ds(g, L)], 3)

    def w_start(c, buf):
        sl = pl.ds(c * WCHUNK, WCHUNK)
        return pltpu.async_copy(wfp_hbm.at[wblk_v.at[sl]], wbuf_v.at[buf],
                                sems.at[buf])

    def w_extract(c, buf):
        @pl.loop(0, WCHUNK, step=L)
        def _(g):
            wv = widx_v[pl.ds(c * WCHUNK + g, L)]
            for j in range(L):
                sub = lax.bitwise_and(wv[j], 7)
                cidx = row_iota + sub * F
                ridx = jnp.full((L,), g + j, jnp.int32)
                wrow = plsc.load_gather(wbuf_v.at[buf], [ridx, cidx])
                wrows_v[pl.ds((c * WCHUNK + g + j) * F, F)] = wrow

    wcp = w_start(0, 0)
    for c in range(n_wchunks):
        nxt = w_start(c + 1, 1 - c % 2) if c + 1 < n_wchunks else None
        wcp.wait()
        w_extract(c, c % 2)
        wcp = nxt

    lane16 = row_iota * F

    @pl.loop(0, b_per_w, step=L)
    def _(p0):
        opos = row_iota + p0
        del opos
        acc = jnp.zeros((L,), jnp.float32)
        for f in range(F):
            tcol = plsc.load_gather(trows_v, [lane16 + (p0 * F + f)])
            wcol = plsc.load_gather(wrows_v, [lane16 + (p0 * F + f)])
            acc = acc + tcol * wcol
        out_v[pl.ds(p0, L)] = 1.0 / (1.0 + jnp.exp(-acc))

    pltpu.sync_copy(out_v, out_hbm.at[pl.ds(base, b_per_w)])


@jax.jit
def _mf_forward(task, worker, task_factors, worker_factors):
    B = task.shape[0]
    b_per_w = B // NW
    tfT = task_factors.T
    wfp = worker_factors.reshape(worker_factors.shape[0] // 8, 8 * F)
    mesh = plsc.VectorSubcoreMesh(core_axis_name="c", subcore_axis_name="s")
    cp = pltpu.CompilerParams(needs_layout_passes=False,
                              use_tc_tiling_on_sc=True)

    task_kern = functools.partial(
        pl.kernel,
        compiler_params=cp,
        out_type=jax.ShapeDtypeStruct((NW, b_per_w * F), jnp.float32),
        mesh=mesh,
        scratch_types=[
            pltpu.VMEM((b_per_w,), jnp.int32),
            pltpu.VMEM((NRING, L, F, 128), jnp.float32),
            pltpu.VMEM((b_per_w * F,), jnp.float32),
            pltpu.SemaphoreType.DMA((NRING,)),
        ],
    )(_task_kernel_body)
    trows = task_kern(task, tfT)

    dot_kern = functools.partial(
        pl.kernel,
        compiler_params=cp,
        out_type=jax.ShapeDtypeStruct((B,), jnp.float32),
        mesh=mesh,
        scratch_types=[
            pltpu.VMEM((b_per_w,), jnp.int32),
            pltpu.VMEM((b_per_w,), jnp.int32),
            pltpu.VMEM((2, WCHUNK, 8 * F), jnp.float32),
            pltpu.VMEM((b_per_w * F,), jnp.float32),
            pltpu.VMEM((b_per_w * F,), jnp.float32),
            pltpu.VMEM((b_per_w,), jnp.float32),
            pltpu.SemaphoreType.DMA((2,)),
        ],
    )(_dot_kernel_body)
    return dot_kern(worker, wfp, trows)


def kernel(task, worker, task_factors, worker_factors):
    return _mf_forward(task, worker, task_factors, worker_factors)


# speedup vs baseline: 1.0066x; 1.0066x over previous
import functools

import jax
import jax.numpy as jnp
from jax import lax
from jax.experimental import pallas as pl
from jax.experimental.pallas import tpu as pltpu
from jax.experimental.pallas import tpu_sc as plsc

NC = 2
NS = 16
NW = NC * NS
L = 16
F = 16
WCHUNK = 64
NRING = 3


def _task_kernel_body(task_hbm, tfT_hbm, trows_hbm,
                      tidx_v, tring_v, trows_v, sems):
    b_per_w = tidx_v.shape[0]
    n_groups = b_per_w // L
    wid = lax.axis_index("s") * NC + lax.axis_index("c")
    base = wid * b_per_w

    pltpu.sync_copy(task_hbm.at[pl.ds(base, b_per_w)], tidx_v)

    row_iota = lax.iota(jnp.int32, L)

    def t_fire(g, ring):
        tv = tidx_v[pl.ds(g * L, L)]
        for j in range(L):
            blk = pl.multiple_of(
                lax.shift_right_logical(tv[j], 7) * 128, 128)
            pltpu.async_copy(tfT_hbm.at[:, pl.ds(blk, 128)],
                             tring_v.at[ring, j], sems.at[ring])

    def t_drain(ring):
        for j in range(L):
            pltpu.make_async_copy(tfT_hbm.at[:, pl.ds(0, 128)],
                                  tring_v.at[ring, j], sems.at[ring]).wait()

    for r in range(NRING):
        t_fire(r, r)

    def t_group(g, ring):
        t_drain(ring)
        tv = tidx_v[pl.ds(g * L, L)]
        for j in range(L):
            col = lax.bitwise_and(tv[j], 127)
            cidx = jnp.full((L,), col, jnp.int32)
            tcol = plsc.load_gather(tring_v.at[ring, j], [row_iota, cidx])
            trows_v[pl.ds((g * L + j) * F, F)] = tcol

        @pl.when(g + NRING < n_groups)
        def _():
            t_fire(g + NRING, ring)

    @pl.loop(0, n_groups, step=NRING)
    def _(g):
        for r in range(NRING):
            @pl.when(g + r < n_groups)
            def _(r=r):
                t_group(g + r, r)

    pltpu.sync_copy(trows_v, trows_hbm.at[wid])


def _dot_kernel_body(worker_hbm, wfp_hbm, trows_hbm, out_hbm,
                     widx_v, wblk_v, wbuf_v, trows_v, wrows_v, out_v, sems):
    b_per_w = widx_v.shape[0]
    n_wchunks = b_per_w // WCHUNK
    wid = lax.axis_index("s") * NC + lax.axis_index("c")
    base = wid * b_per_w

    pltpu.sync_copy(worker_hbm.at[pl.ds(base, b_per_w)], widx_v)
    pltpu.sync_copy(trows_hbm.at[wid], trows_v)

    row_iota = lax.iota(jnp.int32, L)

    @pl.loop(0, b_per_w, step=L)
    def _(g):
        wblk_v[pl.ds(g, L)] = lax.shift_right_logical(widx_v[pl.ds(g, L)], 3)

    def w_start(c, buf):
        sl = pl.ds(c * WCHUNK, WCHUNK)
        return pltpu.async_copy(wfp_hbm.at[wblk_v.at[sl]], wbuf_v.at[buf],
                                sems.at[buf])

    def w_extract(c, buf):
        @pl.loop(0, WCHUNK, step=L)
        def _(g):
            wv = widx_v[pl.ds(c * WCHUNK + g, L)]
            for j in range(L):
                sub = lax.bitwise_and(wv[j], 7)
                cidx = row_iota + sub * F
                ridx = jnp.full((L,), g + j, jnp.int32)
                wrow = plsc.load_gather(wbuf_v.at[buf], [ridx, cidx])
                wrows_v[pl.ds((c * WCHUNK + g + j) * F, F)] = wrow

    wcp = w_start(0, 0)
    for c in range(n_wchunks):
        nxt = w_start(c + 1, 1 - c % 2) if c + 1 < n_wchunks else None
        wcp.wait()
        w_extract(c, c % 2)
        wcp = nxt

    lane16 = row_iota * F

    @pl.loop(0, b_per_w, step=L)
    def _(p0):
        acc = jnp.zeros((L,), jnp.float32)
        for f in range(F):
            tcol = plsc.load_gather(trows_v, [lane16 + (p0 * F + f)])
            wcol = plsc.load_gather(wrows_v, [lane16 + (p0 * F + f)])
            acc = acc + tcol * wcol
        out_v[pl.ds(p0, L)] = 1.0 / (1.0 + jnp.exp(-acc))

    pltpu.sync_copy(out_v, out_hbm.at[pl.ds(base, b_per_w)])


@jax.jit
def _mf_forward(task, worker, task_factors, worker_factors):
    B = task.shape[0]
    b_per_w = B // NW
    tfT = task_factors.T
    wfp = worker_factors.reshape(worker_factors.shape[0] // 8, 8 * F)
    mesh = plsc.VectorSubcoreMesh(core_axis_name="c", subcore_axis_name="s")
    cp = pltpu.CompilerParams(needs_layout_passes=False,
                              use_tc_tiling_on_sc=True)

    task_kern = functools.partial(
        pl.kernel,
        compiler_params=cp,
        out_type=jax.ShapeDtypeStruct((NW, b_per_w * F), jnp.float32),
        mesh=mesh,
        scratch_types=[
            pltpu.VMEM((b_per_w,), jnp.int32),
            pltpu.VMEM((NRING, L, F, 128), jnp.float32),
            pltpu.VMEM((b_per_w * F,), jnp.float32),
            pltpu.SemaphoreType.DMA((NRING,)),
        ],
    )(_task_kernel_body)
    trows = task_kern(task, tfT)

    dot_kern = functools.partial(
        pl.kernel,
        compiler_params=cp,
        out_type=jax.ShapeDtypeStruct((B,), jnp.float32),
        mesh=mesh,
        scratch_types=[
            pltpu.VMEM((b_per_w,), jnp.int32),
            pltpu.VMEM((b_per_w,), jnp.int32),
            pltpu.VMEM((2, WCHUNK, 8 * F), jnp.float32),
            pltpu.VMEM((b_per_w * F,), jnp.float32),
            pltpu.VMEM((b_per_w * F,), jnp.float32),
            pltpu.VMEM((b_per_w,), jnp.float32),
            pltpu.SemaphoreType.DMA((2,)),
        ],
    )(_dot_kernel_body)
    return dot_kern(worker, wfp, trows)


def kernel(task, worker, task_factors, worker_factors):
    return _mf_forward(task, worker, task_factors, worker_factors)
